# Initial kernel scaffold; baseline (speedup 1.0000x reference)
import jax
import jax.numpy as jnp
from jax.experimental import pallas as pl


def kernel(x, edge_list, batch, W1, b1, W_mid, b_mid, W_lin, b_lin):
    raise NotImplementedError("write your pallas kernel here")


# speedup vs baseline: 31.4292x; 31.4292x over previous
import functools

import jax
import jax.numpy as jnp
from jax import lax
from jax.experimental import pallas as pl
from jax.experimental.pallas import tpu as pltpu
from jax.experimental.pallas import tpu_sc as plsc

N = 10000
NPAD = 10112
E = 320000
HID = 64
NG = 64
NF = 128
ODIM = 10

NC = 2
NS = 16
NW = NC * NS
CHUNK = 128
KCH = 80
EPT = CHUNK * KCH
EPAD = EPT * NW
SLICE = NPAD // NS

_mesh = plsc.VectorSubcoreMesh(core_axis_name="c", subcore_axis_name="s")


@functools.partial(
    pl.kernel,
    out_type=jax.ShapeDtypeStruct((NC, NPAD, HID), jnp.float32),
    mesh=_mesh,
    scratch_types=[
        pltpu.VMEM_SHARED((NPAD, HID), jnp.float32),
        pltpu.VMEM((KCH, CHUNK), jnp.int32),
        pltpu.VMEM((KCH, CHUNK), jnp.int32),
        [pltpu.VMEM((CHUNK, HID), jnp.float32)] * 4,
        [pltpu.SemaphoreType.DMA] * 4,
        [pltpu.SemaphoreType.DMA] * 4,
    ],
    compiler_params=pltpu.CompilerParams(use_tc_tiling_on_sc=False),
)
def _sc_spmm(p_hbm, init_hbm, row2_hbm, col2_hbm, out_hbm,
             acc_sh, row_v, col_v, bufs, gsems, ssems):
    c = lax.axis_index("c")
    s = lax.axis_index("s")
    w = c * NS + s
    sl = pl.ds(s * SLICE, SLICE)

    @pl.when(c == 0)
    def _():
        pltpu.async_copy(p_hbm.at[sl], acc_sh.at[sl], ssems[0])

    @pl.when(c != 0)
    def _():
        pltpu.async_copy(init_hbm.at[sl], acc_sh.at[sl], ssems[0])

    pltpu.async_copy(row2_hbm.at[pl.ds(w * KCH, KCH)], row_v, ssems[1])
    pltpu.async_copy(col2_hbm.at[pl.ds(w * KCH, KCH)], col_v, ssems[2])
    pltpu.make_async_copy(row2_hbm.at[pl.ds(w * KCH, KCH)], row_v,
                          ssems[1]).wait()
    pltpu.make_async_copy(col2_hbm.at[pl.ds(w * KCH, KCH)], col_v,
                          ssems[2]).wait()
    pltpu.async_copy(p_hbm.at[row_v.at[0]], bufs[0], gsems[0])
    pltpu.async_copy(p_hbm.at[row_v.at[1]], bufs[1], gsems[1])
    pltpu.make_async_copy(p_hbm.at[sl], acc_sh.at[sl], ssems[0]).wait()
    plsc.subcore_barrier()

    NB = 4

    def body(k, carry):
        for b in range(NB):
            j = NB * k + b
            bs = (b - 2) % NB

            @pl.when(j >= NB)
            def _():
                pltpu.make_async_copy(
                    bufs[b], acc_sh.at[col_v.at[j - NB]], ssems[b]).wait()

            @pl.when(j >= 2)
            def _():
                pltpu.async_copy(p_hbm.at[row_v.at[j]], bufs[b], gsems[b])

            @pl.when(j >= 2)
            def _():
                i = j - 2
                pltpu.make_async_copy(
                    p_hbm.at[row_v.at[i]], bufs[bs], gsems[bs]).wait()
                pltpu.async_copy(bufs[bs], acc_sh.at[col_v.at[i]],
                                 ssems[bs], add=True)
        return carry

    lax.fori_loop(0, KCH // NB, body, 0)
    for i in (KCH - 2, KCH - 1):
        b = i % NB
        pltpu.make_async_copy(p_hbm.at[row_v.at[i]], bufs[b], gsems[b]).wait()
        pltpu.async_copy(bufs[b], acc_sh.at[col_v.at[i]], ssems[b], add=True)
    for b in range(NB):
        i = KCH - NB + b
        pltpu.make_async_copy(bufs[b], acc_sh.at[col_v.at[i]], ssems[b]).wait()
    plsc.subcore_barrier()
    pltpu.sync_copy(acc_sh.at[sl], out_hbm.at[c, sl])


def _tc_prep_body(deg2_ref, x_ref, w1_ref, p_ref, g_ref):
    deg = deg2_ref[0, :, 0:1] + deg2_ref[1, :, 0:1]
    rows = lax.broadcasted_iota(jnp.int32, (NPAD, 1), 0)
    g = jnp.where(rows < N, lax.rsqrt(deg), 0.0)
    g_ref[...] = g
    xw = jnp.dot(x_ref[...], w1_ref[...], preferred_element_type=jnp.float32)
    p_ref[0:N, :] = g[0:N, :] * xw
    p_ref[N:NPAD, :] = jnp.zeros((NPAD - N, HID), jnp.float32)


_tc_prep = pl.pallas_call(
    _tc_prep_body,
    out_shape=(jax.ShapeDtypeStruct((NPAD, HID), jnp.float32),
               jax.ShapeDtypeStruct((NPAD, 1), jnp.float32)),
)


def _tc_mid_body(s2_ref, g_ref, b_ref, w_ref, p_ref):
    g = g_ref[...]
    h = jnp.maximum(g * (s2_ref[0] + s2_ref[1]) + b_ref[...], 0.0)
    p_ref[...] = g * jnp.dot(h, w_ref[...], preferred_element_type=jnp.float32)


_tc_mid = pl.pallas_call(
    _tc_mid_body,
    out_shape=jax.ShapeDtypeStruct((NPAD, HID), jnp.float32),
)


def _tc_final_body(s2_ref, g_ref, b_ref, batch_ref, wl_ref, bl_ref, o_ref):
    g = g_ref[...]
    h = jnp.maximum(g * (s2_ref[0] + s2_ref[1]) + b_ref[...], 0.0)
    gids = lax.broadcasted_iota(jnp.int32, (NG, NPAD), 0)
    oh = (gids == batch_ref[...]).astype(jnp.float32)
    sums = jnp.dot(oh, h, preferred_element_type=jnp.float32)
    cnts = jnp.sum(oh, axis=1, keepdims=True)
    pooled = sums / jnp.maximum(cnts, 1.0)
    o_ref[...] = (jnp.dot(pooled, wl_ref[...], preferred_element_type=jnp.float32)
                  + bl_ref[...])


_tc_final = pl.pallas_call(
    _tc_final_body,
    out_shape=jax.ShapeDtypeStruct((NG, ODIM), jnp.float32),
)


def kernel(x, edge_list, batch, W1, b1, W_mid, b_mid, W_lin, b_lin):
    row = edge_list[0].astype(jnp.int32)
    col = edge_list[1].astype(jnp.int32)
    pad = (N + jnp.arange(EPAD - E, dtype=jnp.int32) % (NPAD - N))
    row2 = jnp.concatenate([row, pad]).reshape(NW * KCH, CHUNK)
    col2 = jnp.concatenate([col, pad]).reshape(NW * KCH, CHUNK)
    ones_h = jnp.ones((NPAD, HID), jnp.float32)
    zeros_h = jnp.zeros((NPAD, HID), jnp.float32)
    batch_p = jnp.concatenate(
        [batch.astype(jnp.int32), jnp.full((NPAD - N,), NG + 7, jnp.int32)]
    ).reshape(1, NPAD)

    deg2 = _sc_spmm(ones_h, zeros_h, row2, col2)
    P, g = _tc_prep(deg2, x, W1)

    b_prev = b1.reshape(1, HID)
    for i in range(W_mid.shape[0]):
        S2 = _sc_spmm(P, zeros_h, row2, col2)
        P = _tc_mid(S2, g, b_prev, W_mid[i])
        b_prev = b_mid[i].reshape(1, HID)

    S2 = _sc_spmm(P, zeros_h, row2, col2)
    return _tc_final(S2, g, b_prev, batch_p, W_lin, b_lin.reshape(1, ODIM))
